# Initial kernel scaffold; baseline (speedup 1.0000x reference)
import jax
import jax.numpy as jnp
from jax.experimental import pallas as pl


def kernel(hidden_states, q_lora, positions, wq_b, wk, k_norm_w, k_norm_b, w_proj):
    raise NotImplementedError("write your pallas kernel here")


# speedup vs baseline: 1.4939x; 1.4939x over previous
import functools

import jax
import jax.numpy as jnp
from jax import lax
from jax.experimental import pallas as pl
from jax.experimental.pallas import tpu as pltpu

_T = 2048
_DM = 2048
_RQ = 1536
_H = 32
_D = 128
_R = 64
_TOPK = 512
_EPS = 1e-6


def _rope_apply(x, c128, s128, nheads):
    if nheads > 1:
        c = jnp.concatenate([c128] * nheads, axis=-1)
        s = jnp.concatenate([s128] * nheads, axis=-1)
    else:
        c, s = c128, s128
    lane = lax.broadcasted_iota(jnp.int32, x.shape, 1) % 128
    swapped = jnp.where(lane < 32, jnp.roll(x, -32, axis=1), jnp.roll(x, 32, axis=1))
    return x * c + swapped * s


def _logits_kernel(q_ref, k_ref, hs_ref, wproj_ref, c_ref, s_ref, ck_ref, sk_ref,
                   out_ref):
    bt = q_ref.shape[0]
    t0 = pl.program_id(0) * bt
    qr = _rope_apply(q_ref[...], c_ref[...], s_ref[...], _H)
    kr = _rope_apply(k_ref[...], ck_ref[...], sk_ref[...], 1)
    scale = (float(_D) ** -0.5) * (float(_H) ** -0.5)
    w = jax.lax.dot_general(hs_ref[...], wproj_ref[...], (((1,), (0,)), ((), ())),
                            preferred_element_type=jnp.float32) * scale
    acc = jnp.zeros((bt, _T), dtype=jnp.float32)
    for h in range(_H):
        qh = qr[:, h * _D:(h + 1) * _D]
        s = jax.lax.dot_general(qh, kr, (((1,), (1,)), ((), ())),
                                preferred_element_type=jnp.float32)
        acc = acc + w[:, h:h + 1] * jnp.maximum(s, 0.0)
    row = t0 + lax.broadcasted_iota(jnp.int32, (bt, _T), 0)
    col = lax.broadcasted_iota(jnp.int32, (bt, _T), 1)
    out_ref[...] = jnp.where(row >= col, acc, -jnp.inf)


def kernel(hidden_states, q_lora, positions, wq_b, wk, k_norm_w, k_norm_b, w_proj):
    q = jnp.matmul(q_lora, wq_b)
    k = jnp.matmul(hidden_states, wk)
    mu = jnp.mean(k, axis=-1, keepdims=True)
    var = jnp.mean((k - mu) ** 2, axis=-1, keepdims=True)
    k = (k - mu) / jnp.sqrt(var + _EPS) * k_norm_w + k_norm_b

    posf = positions.astype(jnp.float32)
    inv_freq = 1.0 / (10000.0 ** (jnp.arange(0, _R, 2, dtype=jnp.float32) / _R))
    ang = posf[:, None] * inv_freq[None, :]
    cos, sin = jnp.cos(ang), jnp.sin(ang)
    ones = jnp.ones((_T, 64), jnp.float32)
    zeros = jnp.zeros((_T, 64), jnp.float32)
    c128 = jnp.concatenate([cos, cos, ones], axis=-1)
    s128 = jnp.concatenate([-sin, sin, zeros], axis=-1)

    bt = 256
    logits = pl.pallas_call(
        _logits_kernel,
        grid=(_T // bt,),
        in_specs=[
            pl.BlockSpec((bt, _H * _D), lambda i: (i, 0)),
            pl.BlockSpec((_T, _D), lambda i: (0, 0)),
            pl.BlockSpec((bt, _DM), lambda i: (i, 0)),
            pl.BlockSpec((_DM, _H), lambda i: (0, 0)),
            pl.BlockSpec((bt, _D), lambda i: (i, 0)),
            pl.BlockSpec((bt, _D), lambda i: (i, 0)),
            pl.BlockSpec((_T, _D), lambda i: (0, 0)),
            pl.BlockSpec((_T, _D), lambda i: (0, 0)),
        ],
        out_specs=pl.BlockSpec((bt, _T), lambda i: (i, 0)),
        out_shape=jax.ShapeDtypeStruct((_T, _T), jnp.float32),
    )(q, k, hidden_states, w_proj, c128, s128, c128, s128)

    _, topk_idx = lax.top_k(logits, _TOPK)
    return topk_idx


# speedup vs baseline: 1.7411x; 1.1655x over previous
import functools

import jax
import jax.numpy as jnp
from jax import lax
from jax.experimental import pallas as pl
from jax.experimental.pallas import tpu as pltpu
from jax.experimental.pallas import tpu_sc as plsc

_T = 2048
_DM = 2048
_RQ = 1536
_H = 32
_D = 128
_R = 64
_TOPK = 512
_EPS = 1e-6


def _rope_apply(x, c128, s128, nheads):
    if nheads > 1:
        c = jnp.concatenate([c128] * nheads, axis=-1)
        s = jnp.concatenate([s128] * nheads, axis=-1)
    else:
        c, s = c128, s128
    lane = lax.broadcasted_iota(jnp.int32, x.shape, 1) % 128
    swapped = jnp.where(lane < 32, jnp.roll(x, -32, axis=1), jnp.roll(x, 32, axis=1))
    return x * c + swapped * s


def _logits_kernel(q_ref, k_ref, hs_ref, wproj_ref, c_ref, s_ref, ck_ref, sk_ref,
                   out_ref):
    bt = q_ref.shape[0]
    t0 = pl.program_id(0) * bt
    qr = _rope_apply(q_ref[...], c_ref[...], s_ref[...], _H)
    kr = _rope_apply(k_ref[...], ck_ref[...], sk_ref[...], 1)
    scale = (float(_D) ** -0.5) * (float(_H) ** -0.5)
    w = jax.lax.dot_general(hs_ref[...], wproj_ref[...], (((1,), (0,)), ((), ())),
                            preferred_element_type=jnp.float32) * scale
    acc = jnp.zeros((bt, _T), dtype=jnp.float32)
    for h in range(_H):
        qh = qr[:, h * _D:(h + 1) * _D]
        s = jax.lax.dot_general(qh, kr, (((1,), (1,)), ((), ())),
                                preferred_element_type=jnp.float32)
        acc = acc + w[:, h:h + 1] * jnp.maximum(s, 0.0)
    row = t0 + lax.broadcasted_iota(jnp.int32, (bt, _T), 0)
    col = lax.broadcasted_iota(jnp.int32, (bt, _T), 1)
    b = lax.bitcast_convert_type(acc, jnp.int32)
    m = (b >> 31) | jnp.int32(-2147483648)
    key = lax.bitcast_convert_type(b ^ m, jnp.uint32)
    masked = (jnp.int32(_T - 1) - col).astype(jnp.uint32)
    out_ref[...] = jnp.where(row >= col, key, masked)


_NW = 32
_ROWS_PER = _T // _NW


def _sc_topk_kernel(keys_hbm, out_hbm, ka, ia, kb, ib, sem):
    wid = lax.axis_index("s") * 2 + lax.axis_index("c")
    iota16 = lax.iota(jnp.int32, 16)

    def vs(buf, i):
        return buf[pl.ds(i * 16, 16)]

    def merge_bitonic(kbuf, ibuf, base, nv):
        D = nv // 2
        while D >= 1:
            for blk in range(0, nv, 2 * D):
                for j in range(D):
                    i0 = base + blk + j
                    i1 = i0 + D
                    a_k, b_k = vs(kbuf, i0), vs(kbuf, i1)
                    a_i, b_i = vs(ibuf, i0), vs(ibuf, i1)
                    cm = a_k >= b_k
                    kbuf[pl.ds(i0 * 16, 16)] = jnp.where(cm, a_k, b_k)
                    kbuf[pl.ds(i1 * 16, 16)] = jnp.where(cm, b_k, a_k)
                    ibuf[pl.ds(i0 * 16, 16)] = jnp.where(cm, a_i, b_i)
                    ibuf[pl.ds(i1 * 16, 16)] = jnp.where(cm, b_i, a_i)
            D //= 2
        for j in range(nv):
            kk, vv = plsc.sort_key_val(vs(kbuf, base + j), vs(ibuf, base + j),
                                       descending=True)
            kbuf[pl.ds((base + j) * 16, 16)] = kk
            ibuf[pl.ds((base + j) * 16, 16)] = vv

    def merge_runs(src_k, src_i, dst_k, dst_i, base, nv, top_only):
        for i in range(nv):
            a_k, a_i = vs(src_k, base + i), vs(src_i, base + i)
            rj = base + 2 * nv - 1 - i
            b_k = lax.rev(vs(src_k, rj), (0,))
            b_i = lax.rev(vs(src_i, rj), (0,))
            cm = a_k >= b_k
            dst_k[pl.ds((base + i) * 16, 16)] = jnp.where(cm, a_k, b_k)
            dst_i[pl.ds((base + i) * 16, 16)] = jnp.where(cm, a_i, b_i)
            if not top_only:
                dst_k[pl.ds((base + nv + i) * 16, 16)] = jnp.where(cm, b_k, a_k)
                dst_i[pl.ds((base + nv + i) * 16, 16)] = jnp.where(cm, b_i, a_i)
        merge_bitonic(dst_k, dst_i, base, nv)
        if not top_only:
            merge_bitonic(dst_k, dst_i, base + nv, nv)

    def row_body(r, _):
        row = wid * _ROWS_PER + r
        pltpu.sync_copy(keys_hbm.at[row], ka)

        def sort16(i, _):
            kk, vv = plsc.sort_key_val(ka[pl.ds(i * 16, 16)], iota16 + i * 16,
                                       descending=True)
            ka[pl.ds(i * 16, 16)] = kk
            ia[pl.ds(i * 16, 16)] = vv
            return 0

        lax.fori_loop(0, 128, sort16, 0, unroll=8)

        bufs = ((ka, ia), (kb, ib))
        cur = 0
        for nv in (1, 2, 4, 8, 16):
            src_k, src_i = bufs[cur]
            dst_k, dst_i = bufs[1 - cur]
            npairs = 128 // (2 * nv)

            def pair_body(p, _, src_k=src_k, src_i=src_i, dst_k=dst_k,
                          dst_i=dst_i, nv=nv):
                merge_runs(src_k, src_i, dst_k, dst_i, p * 2 * nv, nv, False)
                return 0

            lax.fori_loop(0, npairs, pair_body, 0)
            cur = 1 - cur

        src_k, src_i = bufs[cur]
        dst_k, dst_i = bufs[1 - cur]

        def trunc_body(p, _):
            for i in range(32):
                a_k, a_i = vs(src_k, p * 64 + i), vs(src_i, p * 64 + i)
                rj = p * 64 + 63 - i
                b_k = lax.rev(vs(src_k, rj), (0,))
                b_i = lax.rev(vs(src_i, rj), (0,))
                cm = a_k >= b_k
                dst_k[pl.ds((p * 64 + i) * 16, 16)] = jnp.where(cm, a_k, b_k)
                dst_i[pl.ds((p * 64 + i) * 16, 16)] = jnp.where(cm, a_i, b_i)
            return 0

        lax.fori_loop(0, 2, trunc_body, 0)
        merge_bitonic(dst_k, dst_i, 0, 32)
        merge_bitonic(dst_k, dst_i, 64, 32)
        fin_k, fin_i = src_k, src_i
        for i in range(32):
            a_k, a_i = vs(dst_k, i), vs(dst_i, i)
            rj = 95 - i
            b_k = lax.rev(vs(dst_k, rj), (0,))
            b_i = lax.rev(vs(dst_i, rj), (0,))
            cm = a_k >= b_k
            fin_k[pl.ds(i * 16, 16)] = jnp.where(cm, a_k, b_k)
            fin_i[pl.ds(i * 16, 16)] = jnp.where(cm, a_i, b_i)
        merge_bitonic(fin_k, fin_i, 0, 32)

        pltpu.sync_copy(fin_i.at[pl.ds(0, _TOPK)], out_hbm.at[row])
        return 0

    lax.fori_loop(0, _ROWS_PER, row_body, 0)


@functools.partial(
    pl.kernel,
    mesh=plsc.VectorSubcoreMesh(core_axis_name="c", subcore_axis_name="s"),
    out_type=jax.ShapeDtypeStruct((_T, _TOPK), jnp.int32),
    compiler_params=pltpu.CompilerParams(needs_layout_passes=False),
    scratch_types=[
        pltpu.VMEM((_T,), jnp.uint32),
        pltpu.VMEM((_T,), jnp.int32),
        pltpu.VMEM((_T,), jnp.uint32),
        pltpu.VMEM((_T,), jnp.int32),
        pltpu.SemaphoreType.DMA,
    ],
)
def _sc_topk(keys_hbm, out_hbm, ka, ia, kb, ib, sem):
    _sc_topk_kernel(keys_hbm, out_hbm, ka, ia, kb, ib, sem)


def kernel(hidden_states, q_lora, positions, wq_b, wk, k_norm_w, k_norm_b, w_proj):
    q = jnp.matmul(q_lora, wq_b)
    k = jnp.matmul(hidden_states, wk)
    mu = jnp.mean(k, axis=-1, keepdims=True)
    var = jnp.mean((k - mu) ** 2, axis=-1, keepdims=True)
    k = (k - mu) / jnp.sqrt(var + _EPS) * k_norm_w + k_norm_b

    posf = positions.astype(jnp.float32)
    inv_freq = 1.0 / (10000.0 ** (jnp.arange(0, _R, 2, dtype=jnp.float32) / _R))
    ang = posf[:, None] * inv_freq[None, :]
    cos, sin = jnp.cos(ang), jnp.sin(ang)
    ones = jnp.ones((_T, 64), jnp.float32)
    zeros = jnp.zeros((_T, 64), jnp.float32)
    c128 = jnp.concatenate([cos, cos, ones], axis=-1)
    s128 = jnp.concatenate([-sin, sin, zeros], axis=-1)

    bt = 256
    keys = pl.pallas_call(
        _logits_kernel,
        grid=(_T // bt,),
        in_specs=[
            pl.BlockSpec((bt, _H * _D), lambda i: (i, 0)),
            pl.BlockSpec((_T, _D), lambda i: (0, 0)),
            pl.BlockSpec((bt, _DM), lambda i: (i, 0)),
            pl.BlockSpec((_DM, _H), lambda i: (0, 0)),
            pl.BlockSpec((bt, _D), lambda i: (i, 0)),
            pl.BlockSpec((bt, _D), lambda i: (i, 0)),
            pl.BlockSpec((_T, _D), lambda i: (0, 0)),
            pl.BlockSpec((_T, _D), lambda i: (0, 0)),
        ],
        out_specs=pl.BlockSpec((bt, _T), lambda i: (i, 0)),
        out_shape=jax.ShapeDtypeStruct((_T, _T), jnp.uint32),
    )(q, k, hidden_states, w_proj, c128, s128, c128, s128)

    return _sc_topk(keys)


# speedup vs baseline: 2.5457x; 1.4621x over previous
import functools

import jax
import jax.numpy as jnp
from jax import lax
from jax.experimental import pallas as pl
from jax.experimental.pallas import tpu as pltpu
from jax.experimental.pallas import tpu_sc as plsc

_T = 2048
_DM = 2048
_RQ = 1536
_H = 32
_D = 128
_R = 64
_TOPK = 512
_EPS = 1e-6


def _rope_apply(x, c128, s128, nheads):
    if nheads > 1:
        c = jnp.concatenate([c128] * nheads, axis=-1)
        s = jnp.concatenate([s128] * nheads, axis=-1)
    else:
        c, s = c128, s128
    lane = lax.broadcasted_iota(jnp.int32, x.shape, 1) % 128
    swapped = jnp.where(lane < 32, jnp.roll(x, -32, axis=1), jnp.roll(x, 32, axis=1))
    return x * c + swapped * s


def _logits_kernel(q_ref, k_ref, hs_ref, wproj_ref, c_ref, s_ref, ck_ref, sk_ref,
                   out_ref):
    bt = q_ref.shape[0]
    t0 = pl.program_id(0) * bt
    qr = _rope_apply(q_ref[...], c_ref[...], s_ref[...], _H)
    kr = _rope_apply(k_ref[...], ck_ref[...], sk_ref[...], 1)
    scale = (float(_D) ** -0.5) * (float(_H) ** -0.5)
    w = jax.lax.dot_general(hs_ref[...], wproj_ref[...], (((1,), (0,)), ((), ())),
                            preferred_element_type=jnp.float32) * scale
    acc = jnp.zeros((bt, _T), dtype=jnp.float32)
    for h in range(_H):
        qh = qr[:, h * _D:(h + 1) * _D]
        s = jax.lax.dot_general(qh, kr, (((1,), (1,)), ((), ())),
                                preferred_element_type=jnp.float32)
        acc = acc + w[:, h:h + 1] * jnp.maximum(s, 0.0)
    row = t0 + lax.broadcasted_iota(jnp.int32, (bt, _T), 0)
    col = lax.broadcasted_iota(jnp.int32, (bt, _T), 1)
    b = lax.bitcast_convert_type(acc, jnp.int32)
    m = (b >> 31) | jnp.int32(-2147483648)
    key = lax.bitcast_convert_type(b ^ m, jnp.uint32)
    masked = (jnp.int32(_T - 1) - col).astype(jnp.uint32)
    out_ref[...] = jnp.where(row >= col, key, masked)


_NW = 32
_ROWS_PER = _T // _NW


def _sc_topk_kernel(keys_hbm, out_hbm, ka, ia, kb, ib, sem):
    wid = lax.axis_index("s") * 2 + lax.axis_index("c")
    iota16 = lax.iota(jnp.int32, 16)

    def vs(buf, i):
        return buf[pl.ds(i * 16, 16)]

    def st(buf, i, x):
        buf[pl.ds(i * 16, 16)] = x

    def bitonic_reg(ks, vi):
        nv = len(ks)
        D = nv // 2
        while D >= 1:
            for blk in range(0, nv, 2 * D):
                for j in range(D):
                    i0, i1 = blk + j, blk + j + D
                    cm = ks[i0] >= ks[i1]
                    hk = jnp.where(cm, ks[i0], ks[i1])
                    lk = jnp.where(cm, ks[i1], ks[i0])
                    hi = jnp.where(cm, vi[i0], vi[i1])
                    li = jnp.where(cm, vi[i1], vi[i0])
                    ks[i0], ks[i1] = hk, lk
                    vi[i0], vi[i1] = hi, li
            D //= 2
        for j in range(nv):
            ks[j], vi[j] = plsc.sort_key_val(ks[j], vi[j], descending=True)

    def merge_runs_reg(src_k, src_i, dst_k, dst_i, base, nv, top_only=False):
        ak = [vs(src_k, base + i) for i in range(nv)]
        ai = [vs(src_i, base + i) for i in range(nv)]
        bk = [lax.rev(vs(src_k, base + 2 * nv - 1 - i), (0,)) for i in range(nv)]
        bi = [lax.rev(vs(src_i, base + 2 * nv - 1 - i), (0,)) for i in range(nv)]
        uk, ui, vk_, vi_ = [], [], [], []
        for i in range(nv):
            cm = ak[i] >= bk[i]
            uk.append(jnp.where(cm, ak[i], bk[i]))
            ui.append(jnp.where(cm, ai[i], bi[i]))
            if not top_only:
                vk_.append(jnp.where(cm, bk[i], ak[i]))
                vi_.append(jnp.where(cm, bi[i], ai[i]))
        bitonic_reg(uk, ui)
        for i in range(nv):
            st(dst_k, base + i, uk[i])
            st(dst_i, base + i, ui[i])
        if not top_only:
            bitonic_reg(vk_, vi_)
            for i in range(nv):
                st(dst_k, base + nv + i, vk_[i])
                st(dst_i, base + nv + i, vi_[i])

    def merge_bitonic_mem(kbuf, ibuf, base, nv):
        D = nv // 2
        while D >= 4:
            for blk in range(0, nv, 2 * D):
                for j in range(D):
                    i0 = base + blk + j
                    i1 = i0 + D
                    a_k, b_k = vs(kbuf, i0), vs(kbuf, i1)
                    a_i, b_i = vs(ibuf, i0), vs(ibuf, i1)
                    cm = a_k >= b_k
                    st(kbuf, i0, jnp.where(cm, a_k, b_k))
                    st(kbuf, i1, jnp.where(cm, b_k, a_k))
                    st(ibuf, i0, jnp.where(cm, a_i, b_i))
                    st(ibuf, i1, jnp.where(cm, b_i, a_i))
            D //= 2
        for blk in range(0, nv, 8):
            ks = [vs(kbuf, base + blk + j) for j in range(8)]
            vi = [vs(ibuf, base + blk + j) for j in range(8)]
            bitonic_reg(ks, vi)
            for j in range(8):
                st(kbuf, base + blk + j, ks[j])
                st(ibuf, base + blk + j, vi[j])

    def build_chunk(c):
        base32 = c * 32

        def sort16(i, _):
            kk, vv = plsc.sort_key_val(vs(ka, base32 + i), iota16 + (base32 + i) * 16,
                                       descending=True)
            st(ka, base32 + i, kk)
            st(ia, base32 + i, vv)
            return 0

        lax.fori_loop(0, 32, sort16, 0, unroll=4)
        bufs = ((ka, ia), (kb, ib))
        cur = 0
        for nv in (1, 2, 4, 8, 16):
            src_k, src_i = bufs[cur]
            dst_k, dst_i = bufs[1 - cur]
            npairs = 32 // (2 * nv)

            def pair_body(p, _, src_k=src_k, src_i=src_i, dst_k=dst_k,
                          dst_i=dst_i, nv=nv, base32=base32):
                merge_runs_reg(src_k, src_i, dst_k, dst_i, base32 + p * 2 * nv, nv)
                return 0

            lax.fori_loop(0, npairs, pair_body, 0)
            cur = 1 - cur
        for i in range(32):
            st(ka, base32 + i, vs(kb, base32 + i))
            st(ia, base32 + i, vs(ib, base32 + i))

    def trunc_merge_into_top(c):
        for i in range(32):
            a_k, a_i = vs(kb, i), vs(ib, i)
            rj = c * 32 + 31 - i
            b_k = lax.rev(vs(ka, rj), (0,))
            b_i = lax.rev(vs(ia, rj), (0,))
            cm = a_k >= b_k
            st(kb, i, jnp.where(cm, a_k, b_k))
            st(ib, i, jnp.where(cm, a_i, b_i))
        merge_bitonic_mem(kb, ib, 0, 32)

    def row_body(r, _):
        row = wid + _NW * r
        nch = jnp.minimum(row // _TOPK + 1, jnp.int32(4))

        def chunk_body(c, _):
            pltpu.sync_copy(keys_hbm.at[row, pl.ds(c * _TOPK, _TOPK)],
                            ka.at[pl.ds(c * _TOPK, _TOPK)])
            build_chunk(c)
            return 0

        lax.fori_loop(0, nch, chunk_body, 0)
        for i in range(32):
            st(kb, i, vs(ka, i))
            st(ib, i, vs(ia, i))

        def merge_body(c, _):
            trunc_merge_into_top(c)
            return 0

        lax.fori_loop(1, nch, merge_body, 0)
        pltpu.sync_copy(ib.at[pl.ds(0, _TOPK)], out_hbm.at[row])
        return 0

    lax.fori_loop(0, _ROWS_PER, row_body, 0)


@functools.partial(
    pl.kernel,
    mesh=plsc.VectorSubcoreMesh(core_axis_name="c", subcore_axis_name="s"),
    out_type=jax.ShapeDtypeStruct((_T, _TOPK), jnp.int32),
    compiler_params=pltpu.CompilerParams(needs_layout_passes=False),
    scratch_types=[
        pltpu.VMEM((_T,), jnp.uint32),
        pltpu.VMEM((_T,), jnp.int32),
        pltpu.VMEM((_T,), jnp.uint32),
        pltpu.VMEM((_T,), jnp.int32),
        pltpu.SemaphoreType.DMA,
    ],
)
def _sc_topk(keys_hbm, out_hbm, ka, ia, kb, ib, sem):
    _sc_topk_kernel(keys_hbm, out_hbm, ka, ia, kb, ib, sem)


def kernel(hidden_states, q_lora, positions, wq_b, wk, k_norm_w, k_norm_b, w_proj):
    q = jnp.matmul(q_lora, wq_b)
    k = jnp.matmul(hidden_states, wk)
    mu = jnp.mean(k, axis=-1, keepdims=True)
    var = jnp.mean((k - mu) ** 2, axis=-1, keepdims=True)
    k = (k - mu) / jnp.sqrt(var + _EPS) * k_norm_w + k_norm_b

    posf = positions.astype(jnp.float32)
    inv_freq = 1.0 / (10000.0 ** (jnp.arange(0, _R, 2, dtype=jnp.float32) / _R))
    ang = posf[:, None] * inv_freq[None, :]
    cos, sin = jnp.cos(ang), jnp.sin(ang)
    ones = jnp.ones((_T, 64), jnp.float32)
    zeros = jnp.zeros((_T, 64), jnp.float32)
    c128 = jnp.concatenate([cos, cos, ones], axis=-1)
    s128 = jnp.concatenate([-sin, sin, zeros], axis=-1)

    bt = 256
    keys = pl.pallas_call(
        _logits_kernel,
        grid=(_T // bt,),
        in_specs=[
            pl.BlockSpec((bt, _H * _D), lambda i: (i, 0)),
            pl.BlockSpec((_T, _D), lambda i: (0, 0)),
            pl.BlockSpec((bt, _DM), lambda i: (i, 0)),
            pl.BlockSpec((_DM, _H), lambda i: (0, 0)),
            pl.BlockSpec((bt, _D), lambda i: (i, 0)),
            pl.BlockSpec((bt, _D), lambda i: (i, 0)),
            pl.BlockSpec((_T, _D), lambda i: (0, 0)),
            pl.BlockSpec((_T, _D), lambda i: (0, 0)),
        ],
        out_specs=pl.BlockSpec((bt, _T), lambda i: (i, 0)),
        out_shape=jax.ShapeDtypeStruct((_T, _T), jnp.uint32),
    )(q, k, hidden_states, w_proj, c128, s128, c128, s128)

    return _sc_topk(keys)
